# Initial kernel scaffold; baseline (speedup 1.0000x reference)
import jax
import jax.numpy as jnp
from jax.experimental import pallas as pl


def kernel(inputs, token_table, position_table):
    raise NotImplementedError("write your pallas kernel here")


# speedup vs baseline: 2.2227x; 2.2227x over previous
import functools

import jax
import jax.numpy as jnp
from jax import lax
from jax.experimental import pallas as pl
from jax.experimental.pallas import tpu as pltpu
from jax.experimental.pallas import tpu_sc as plsc

SEQ_LEN = 200
VOCAB = 100000
DIM = 64
BATCH = 4096

NC = 2
NS = 16
LANES = 16
NW = NC * NS

TOTAL = BATCH * SEQ_LEN
ROWS_PER_W = TOTAL // NW
CHUNK = 256
SUB = 128
N_SUB = CHUNK // SUB
N_CHUNKS = ROWS_PER_W // CHUNK
POS_TILED = SEQ_LEN + CHUNK


def _body(idx_hbm, token_hbm, pos_hbm, out_hbm, idx_v, rows_v, pos_v, sem):
    c = lax.axis_index("c")
    s = lax.axis_index("s")
    wid = s * NC + c
    base = wid * ROWS_PER_W

    pltpu.sync_copy(pos_hbm, pos_v.at[pl.ds(0, SEQ_LEN)])
    pltpu.sync_copy(pos_hbm, pos_v.at[pl.ds(SEQ_LEN, SEQ_LEN)])
    pltpu.sync_copy(pos_hbm.at[pl.ds(0, POS_TILED - 2 * SEQ_LEN)],
                    pos_v.at[pl.ds(2 * SEQ_LEN, POS_TILED - 2 * SEQ_LEN)])

    def chunk_body(i, carry):
        off = base + i * CHUNK
        p0 = lax.rem(i * CHUNK, SEQ_LEN)
        pltpu.sync_copy(idx_hbm.at[pl.ds(off, CHUNK)], idx_v)
        for j in range(N_SUB):
            pltpu.async_copy(
                token_hbm.at[idx_v.at[pl.ds(j * SUB, SUB)]],
                rows_v.at[pl.ds(j * SUB, SUB)],
                sem,
            ).wait()

        def row_add(r, carry):
            p = p0 + r
            for g in range(DIM // LANES):
                sl = pl.ds(g * LANES, LANES)
                rows_v[r, sl] = rows_v[r, sl] + pos_v[p, sl]
            return carry

        lax.fori_loop(0, CHUNK, row_add, 0, unroll=2)
        pltpu.sync_copy(rows_v, out_hbm.at[pl.ds(off, CHUNK)])
        return carry

    lax.fori_loop(0, N_CHUNKS, chunk_body, 0)


@jax.jit
def _run(idx_flat, token_table, position_table):
    mesh = plsc.VectorSubcoreMesh(
        core_axis_name="c", subcore_axis_name="s",
        num_cores=NC, num_subcores=NS,
    )
    fn = pl.kernel(
        _body,
        out_type=jax.ShapeDtypeStruct((TOTAL, DIM), jnp.float32),
        mesh=mesh,
        compiler_params=pltpu.CompilerParams(use_tc_tiling_on_sc=False),
        scratch_types=[
            pltpu.VMEM((CHUNK,), jnp.int32),
            pltpu.VMEM((CHUNK, DIM), jnp.float32),
            pltpu.VMEM((POS_TILED, DIM), jnp.float32),
            pltpu.SemaphoreType.DMA,
        ],
    )
    return fn(idx_flat, token_table, position_table)


def kernel(inputs, token_table, position_table):
    idx_flat = inputs.reshape(-1).astype(jnp.int32)
    out = _run(idx_flat, token_table, position_table)
    return out.reshape(BATCH, SEQ_LEN, DIM)


# speedup vs baseline: 4.1471x; 1.8658x over previous
import jax
import jax.numpy as jnp
from jax import lax
from jax.experimental import pallas as pl
from jax.experimental.pallas import tpu as pltpu
from jax.experimental.pallas import tpu_sc as plsc

SEQ_LEN = 200
VOCAB = 100000
DIM = 64
BATCH = 4096

NC = 2
NS = 16
LANES = 16
NW = NC * NS

TOTAL = BATCH * SEQ_LEN
ROWS_PER_W = TOTAL // NW
CHUNK = 2 * SEQ_LEN
N_CHUNKS = ROWS_PER_W // CHUNK
N_PAIRS = N_CHUNKS // 2
SUBS = ((0, 128), (128, 128), (256, 128), (384, 16))
GROUPS = DIM // LANES


def _body(idx_hbm, token_hbm, pos_hbm, out_hbm,
          idx_v, pos_v, rows_a, rows_b, gsem_a, gsem_b, osem_a, osem_b):
    c = lax.axis_index("c")
    s = lax.axis_index("s")
    wid = s * NC + c
    base = wid * ROWS_PER_W

    pltpu.sync_copy(idx_hbm.at[pl.ds(base, ROWS_PER_W)], idx_v)
    pltpu.sync_copy(pos_hbm, pos_v)

    def fire_gathers(i, buf, sem):
        off = i * CHUNK
        for (o, n) in SUBS:
            pltpu.async_copy(
                token_hbm.at[idx_v.at[pl.ds(off + o, n)]],
                buf.at[pl.ds(o, n)], sem)

    def drain(buf, sem):
        pltpu.make_async_copy(out_hbm.at[pl.ds(0, CHUNK)], buf, sem).wait()

    def add_pos(buf):
        def row(r, carry):
            for g in range(GROUPS):
                sl = pl.ds(g * LANES, LANES)
                pv = pos_v[r, sl]
                plsc.addupdate(buf.at[r, sl], pv)
                plsc.addupdate(buf.at[r + SEQ_LEN, sl], pv)
            return carry
        lax.fori_loop(0, SEQ_LEN, row, 0, unroll=4)

    def writeout(i, buf, sem):
        pltpu.async_copy(buf, out_hbm.at[pl.ds(base + i * CHUNK, CHUNK)], sem)

    def pair(g, carry):
        i0 = 2 * g
        i1 = i0 + 1

        @pl.when(g > 0)
        def _():
            drain(rows_a, osem_a)
        fire_gathers(i0, rows_a, gsem_a)

        @pl.when(g > 0)
        def _():
            drain(rows_b, osem_b)
        fire_gathers(i1, rows_b, gsem_b)

        drain(rows_a, gsem_a)
        add_pos(rows_a)
        writeout(i0, rows_a, osem_a)

        drain(rows_b, gsem_b)
        add_pos(rows_b)
        writeout(i1, rows_b, osem_b)
        return carry

    lax.fori_loop(0, N_PAIRS, pair, 0)
    drain(rows_a, osem_a)
    drain(rows_b, osem_b)


@jax.jit
def _run(idx_flat, token_table, position_table):
    mesh = plsc.VectorSubcoreMesh(
        core_axis_name="c", subcore_axis_name="s",
        num_cores=NC, num_subcores=NS,
    )
    fn = pl.kernel(
        _body,
        out_type=jax.ShapeDtypeStruct((TOTAL, DIM), jnp.float32),
        mesh=mesh,
        compiler_params=pltpu.CompilerParams(use_tc_tiling_on_sc=False),
        scratch_types=[
            pltpu.VMEM((ROWS_PER_W,), jnp.int32),
            pltpu.VMEM((SEQ_LEN, DIM), jnp.float32),
            pltpu.VMEM((CHUNK, DIM), jnp.float32),
            pltpu.VMEM((CHUNK, DIM), jnp.float32),
            pltpu.SemaphoreType.DMA,
            pltpu.SemaphoreType.DMA,
            pltpu.SemaphoreType.DMA,
            pltpu.SemaphoreType.DMA,
        ],
    )
    return fn(idx_flat, token_table, position_table)


def kernel(inputs, token_table, position_table):
    idx_flat = inputs.reshape(-1).astype(jnp.int32)
    out = _run(idx_flat, token_table, position_table)
    return out.reshape(BATCH, SEQ_LEN, DIM)
